# baseline (speedup 1.0000x reference)
import functools

import jax
import jax.numpy as jnp
import numpy as np
from jax.experimental import pallas as pl
from jax.experimental.pallas import tpu as pltpu

_EPS_BN = 1e-5
_BN_S = 1.0 / np.sqrt(1.0 + _EPS_BN)
_B, _P, _S1, _S2 = 8, 2048, 410, 103


def _fps_kernel(px_ref, py_ref, pz_ref, px1_ref, py1_ref, pz1_ref,
                px2_ref, py2_ref, pz2_ref):
    px, py, pz = px_ref[...], py_ref[...], pz_ref[...]
    lane_p = jax.lax.broadcasted_iota(jnp.int32, (_B, _P), 1)
    lane_s1 = jax.lax.broadcasted_iota(jnp.int32, (_B, _S1), 1)

    def run_fps(ax_src, ay_src, az_src, n_pts, n_samp, lane_src, lane_dst):
        lx = ax_src[:, 0:1]
        ly = ay_src[:, 0:1]
        lz = az_src[:, 0:1]
        ox = jnp.where(lane_dst == 0, lx, 0.0)
        oy = jnp.where(lane_dst == 0, ly, 0.0)
        oz = jnp.where(lane_dst == 0, lz, 0.0)
        dists = jnp.full((_B, n_pts), 1e30, dtype=jnp.float32)

        def body(i, c):
            lx, ly, lz, dists, ox, oy, oz = c
            d = (ax_src - lx) ** 2 + (ay_src - ly) ** 2 + (az_src - lz) ** 2
            dists = jnp.minimum(dists, d)
            m = jnp.max(dists, axis=1, keepdims=True)
            sel = jnp.min(jnp.where(dists == m, lane_src, n_pts),
                          axis=1, keepdims=True)
            eq = lane_src == sel
            nlx = jnp.sum(jnp.where(eq, ax_src, 0.0), axis=1, keepdims=True)
            nly = jnp.sum(jnp.where(eq, ay_src, 0.0), axis=1, keepdims=True)
            nlz = jnp.sum(jnp.where(eq, az_src, 0.0), axis=1, keepdims=True)
            ox = jnp.where(lane_dst == i, nlx, ox)
            oy = jnp.where(lane_dst == i, nly, oy)
            oz = jnp.where(lane_dst == i, nlz, oz)
            return (nlx, nly, nlz, dists, ox, oy, oz)

        c = (lx, ly, lz, dists, ox, oy, oz)
        c = jax.lax.fori_loop(1, n_samp, body, c)
        return c[4], c[5], c[6]

    px1, py1, pz1 = run_fps(px, py, pz, _P, _S1, lane_p, lane_s1)
    px1_ref[...], py1_ref[...], pz1_ref[...] = px1, py1, pz1
    lane_s2 = jax.lax.broadcasted_iota(jnp.int32, (_B, _S2), 1)
    px2, py2, pz2 = run_fps(px1, py1, pz1, _S1, _S2, lane_s1, lane_s2)
    px2_ref[...], py2_ref[...], pz2_ref[...] = px2, py2, pz2


def _lane_cumsum(x, n):
    s = 1
    while s < n:
        shifted = jnp.concatenate(
            [jnp.zeros((x.shape[0], s), x.dtype), x[:, : n - s]], axis=1)
        x = x + shifted
        s *= 2
    return x


def _make_radius_conv_kernel(N, Q, QP, F, rr, dims, QB, K=64):
    n_layers = len(dims) - 1
    rr_bits = int(np.frombuffer(np.float32(rr).tobytes(), dtype=np.int32)[0])
    SENT = np.int32(2**31 - 1)

    def kern(*refs):
        px_r, py_r, pz_r, x_ref, qx_c, qy_c, qz_c = refs[:7]
        wrefs = refs[7:-1]
        out_ref = refs[-1]
        px = px_r[...].reshape(1, N)
        py = py_r[...].reshape(1, N)
        pz = pz_r[...].reshape(1, N)
        qx = qx_c[...].reshape(QP, 1)
        qy = qy_c[...].reshape(QP, 1)
        qz = qz_c[...].reshape(QP, 1)
        xs = x_ref[...].reshape(N, F)
        W1 = wrefs[0][...]
        b1 = wrefs[1][...]
        d2 = (qx - px) ** 2 + (qy - py) ** 2 + (qz - pz) ** 2
        key = jax.lax.bitcast_convert_type(d2, jnp.int32)
        within = key < rr_bits
        key = jnp.where(within, key, SENT)
        lo = jnp.zeros((QP, 1), jnp.int32)
        hi = jnp.full((QP, 1), SENT, jnp.int32)

        def bs_body(_, c):
            lo, hi = c
            mid = lo + (hi - lo) // 2
            cnt = jnp.sum((key <= mid).astype(jnp.int32), axis=1,
                          keepdims=True)
            ge = cnt >= K
            hi = jnp.where(ge, mid, hi)
            lo = jnp.where(ge, lo, mid + 1)
            return lo, hi

        lo, hi = jax.lax.fori_loop(0, 31, bs_body, (lo, hi))
        v64 = hi
        less = key < v64
        cnt_less = jnp.sum(less.astype(jnp.int32), axis=1, keepdims=True)
        tie = key == v64
        tie_rank = _lane_cumsum(tie.astype(jnp.int32), N) - 1
        sel = within & (less | (tie & (tie_rank < (K - cnt_less))))
        count = jnp.sum(sel.astype(jnp.int32), axis=1, keepdims=True)
        slot = _lane_cumsum(sel.astype(jnp.int32), N) - 1
        slot = jnp.where(sel, slot, -1)
        a = jnp.dot(xs, W1[:F, :], preferred_element_type=jnp.float32)
        a = a + px.reshape(N, 1) * W1[F, :].reshape(1, -1)
        a = a + py.reshape(N, 1) * W1[F + 1, :].reshape(1, -1)
        a = a + pz.reshape(N, 1) * W1[F + 2, :].reshape(1, -1)
        cq = b1 - (qx * W1[F, :].reshape(1, -1)
                   + qy * W1[F + 1, :].reshape(1, -1)
                   + qz * W1[F + 2, :].reshape(1, -1))
        s_iota = jax.lax.broadcasted_iota(jnp.int32, (1, K, 1), 1)
        s_iota_b = s_iota.astype(jnp.bfloat16)
        slot_b = slot.astype(jnp.bfloat16)
        a_hi = a.astype(jnp.bfloat16)
        a_lo = (a - a_hi.astype(jnp.float32)).astype(jnp.bfloat16)
        gam = wrefs[2][...]
        bet = wrefs[3][...]
        for qb in range(QP // QB):
            sl = slot_b[qb * QB:(qb + 1) * QB, :]
            m3 = (sl[:, None, :] == s_iota_b).astype(jnp.bfloat16)
            g = jax.lax.dot_general(
                m3, a_hi, (((2,), (0,)), ((), ())),
                preferred_element_type=jnp.float32)
            g = g + jax.lax.dot_general(
                m3, a_lo, (((2,), (0,)), ((), ())),
                preferred_element_type=jnp.float32)
            h = g + cq[qb * QB:(qb + 1) * QB, None, :]
            h = jnp.maximum(h, 0.0) * (_BN_S * gam) + bet
            for li in range(1, n_layers):
                W = wrefs[4 * li][...]
                b = wrefs[4 * li + 1][...]
                gm = wrefs[4 * li + 2][...]
                bt = wrefs[4 * li + 3][...]
                h = jax.lax.dot_general(
                    h, W, (((2,), (0,)), ((), ())),
                    preferred_element_type=jnp.float32) + b
                h = jnp.maximum(h, 0.0) * (_BN_S * gm) + bt
            cb = count[qb * QB:(qb + 1) * QB, :]
            vmask = s_iota < cb[:, None, :]
            h = jnp.where(vmask, h, -1e30)
            h = jnp.max(h, axis=1)
            h = jnp.where(cb > 0, h, 0.0)
            out_ref[0, qb * QB:(qb + 1) * QB, :] = h

    return kern


def _radius_conv(pb_feat, pb_pos, q_pos, layers, N, Q, rr, QB=16):
    B = pb_feat.shape[0]
    F = pb_feat.shape[2]
    dims = [F + 3] + [l["W"].shape[1] for l in layers]
    QP = -(-Q // QB) * QB
    pad_q = QP - Q
    qp = jnp.pad(q_pos, ((0, 0), (0, pad_q), (0, 0)), constant_values=1e6)
    ins = [pb_pos[:, None, :, 0], pb_pos[:, None, :, 1], pb_pos[:, None, :, 2],
           pb_feat,
           qp[..., 0:1], qp[..., 1:2], qp[..., 2:3]]
    wlist = []
    for l in layers:
        wlist += [l["W"], l["b"].reshape(1, -1), l["gamma"].reshape(1, -1),
                  l["beta"].reshape(1, -1)]
    in_specs = [
        pl.BlockSpec((1, 1, N), lambda i: (i, 0, 0)),
        pl.BlockSpec((1, 1, N), lambda i: (i, 0, 0)),
        pl.BlockSpec((1, 1, N), lambda i: (i, 0, 0)),
        pl.BlockSpec((1, N, F), lambda i: (i, 0, 0)),
        pl.BlockSpec((1, QP, 1), lambda i: (i, 0, 0)),
        pl.BlockSpec((1, QP, 1), lambda i: (i, 0, 0)),
        pl.BlockSpec((1, QP, 1), lambda i: (i, 0, 0)),
    ] + [pl.BlockSpec(w.shape, lambda i: (0, 0)) for w in wlist]

    kern = _make_radius_conv_kernel(N, Q, QP, F, rr, dims, QB)

    out = pl.pallas_call(
        kern,
        grid=(B,),
        in_specs=in_specs,
        out_specs=pl.BlockSpec((1, QP, dims[-1]), lambda i: (i, 0, 0)),
        out_shape=jax.ShapeDtypeStruct((B, QP, dims[-1]), jnp.float32),
    )(*ins, *wlist)
    return out[:, :Q, :]


def _make_knn_mlp_kernel(N, Q, QP, D, S, n_bn, n_lin):
    def kern(*refs):
        sx_r, sy_r, sz_r, f_ref, qx_c, qy_c, qz_c, skip_ref = refs[:8]
        wrefs = refs[8:-1]
        out_ref = refs[-1]
        sx = sx_r[...].reshape(1, N)
        sy = sy_r[...].reshape(1, N)
        sz = sz_r[...].reshape(1, N)
        qx = qx_c[...].reshape(QP, 1)
        qy = qy_c[...].reshape(QP, 1)
        qz = qz_c[...].reshape(QP, 1)
        feat = f_ref[...].reshape(N, D)
        skip = skip_ref[...].reshape(QP, S)
        lane = jax.lax.broadcasted_iota(jnp.int32, (QP, N), 1)
        d2 = (qx - sx) ** 2 + (qy - sy) ** 2 + (qz - sz) ** 2
        xs, ws = [], []
        for _ in range(3):
            m = jnp.min(d2, axis=1, keepdims=True)
            idx = jnp.min(jnp.where(d2 == m, lane, N), axis=1, keepdims=True)
            eq = (lane == idx).astype(jnp.float32)
            xs.append(jnp.dot(eq, feat, preferred_element_type=jnp.float32))
            ws.append(1.0 / jnp.maximum(m, 1e-16))
            d2 = jnp.where(lane == idx, 3e38, d2)
        wsum = (ws[0] + ws[1]) + ws[2]
        h = (xs[0] * (ws[0] / wsum) + xs[1] * (ws[1] / wsum)) \
            + xs[2] * (ws[2] / wsum)
        i = 0
        for li in range(n_bn):
            W, b, g, be = (wrefs[i][...], wrefs[i + 1][...],
                           wrefs[i + 2][...], wrefs[i + 3][...])
            i += 4
            if li == 0:
                z = jnp.dot(h, W[:D, :], preferred_element_type=jnp.float32) \
                    + jnp.dot(skip, W[D:, :],
                              preferred_element_type=jnp.float32) + b
            else:
                z = jnp.dot(h, W, preferred_element_type=jnp.float32) + b
            h = jnp.maximum(z, 0.0) * (_BN_S * g) + be
        for li in range(n_lin):
            W, b = wrefs[i][...], wrefs[i + 1][...]
            i += 2
            h = jnp.dot(h, W, preferred_element_type=jnp.float32) + b
            if li == 0:
                h = jnp.maximum(h, 0.0)
        out_ref[...] = h.reshape(1, QP, h.shape[1])

    return kern


def _knn_mlp(src_pos, dst_pos, feat, skip, bn_layers, lin_layers, N, Q):
    B, D = feat.shape[0], feat.shape[2]
    S = skip.shape[2]
    QP = -(-Q // 8) * 8
    pad_q = QP - Q
    qp = jnp.pad(dst_pos, ((0, 0), (0, pad_q), (0, 0)), constant_values=1e6)
    skp = jnp.pad(skip, ((0, 0), (0, pad_q), (0, 0)))
    ins = [src_pos[:, None, :, 0], src_pos[:, None, :, 1],
           src_pos[:, None, :, 2], feat,
           qp[..., 0:1], qp[..., 1:2], qp[..., 2:3], skp]
    wlist = []
    for l in bn_layers:
        wlist += [l["W"], l["b"].reshape(1, -1), l["gamma"].reshape(1, -1),
                  l["beta"].reshape(1, -1)]
    for l in lin_layers:
        wlist += [l["W"], l["b"].reshape(1, -1)]
    dout = (lin_layers[-1]["W"].shape[1] if lin_layers
            else bn_layers[-1]["W"].shape[1])
    in_specs = [
        pl.BlockSpec((1, 1, N), lambda i: (i, 0, 0)),
        pl.BlockSpec((1, 1, N), lambda i: (i, 0, 0)),
        pl.BlockSpec((1, 1, N), lambda i: (i, 0, 0)),
        pl.BlockSpec((1, N, D), lambda i: (i, 0, 0)),
        pl.BlockSpec((1, QP, 1), lambda i: (i, 0, 0)),
        pl.BlockSpec((1, QP, 1), lambda i: (i, 0, 0)),
        pl.BlockSpec((1, QP, 1), lambda i: (i, 0, 0)),
        pl.BlockSpec((1, QP, S), lambda i: (i, 0, 0)),
    ] + [pl.BlockSpec(w.shape, lambda i: (0, 0)) for w in wlist]
    kern = _make_knn_mlp_kernel(N, Q, QP, D, S, len(bn_layers),
                                len(lin_layers))
    out = pl.pallas_call(
        kern,
        grid=(B,),
        in_specs=in_specs,
        out_specs=pl.BlockSpec((1, QP, dout), lambda i: (i, 0, 0)),
        out_shape=jax.ShapeDtypeStruct((B, QP, dout), jnp.float32),
    )(*ins, *wlist)
    return out[:, :Q, :]


def _fps_pallas(pb):
    px = pb[..., 0]
    py = pb[..., 1]
    pz = pb[..., 2]
    outs = pl.pallas_call(
        _fps_kernel,
        out_shape=[jax.ShapeDtypeStruct((_B, _S1), jnp.float32)] * 3
        + [jax.ShapeDtypeStruct((_B, _S2), jnp.float32)] * 3,
    )(px, py, pz)
    pos1 = jnp.stack(outs[0:3], axis=-1)
    pos2 = jnp.stack(outs[3:6], axis=-1)
    return pos1, pos2


def _mlp(layers, x):
    for l in layers:
        x = x @ l["W"] + l["b"]
        x = jax.nn.relu(x)
        x = x * (_BN_S * l["gamma"]) + l["beta"]
    return x


def _fps(pos, n_sample):
    P = pos.shape[0]
    idxs0 = jnp.zeros((n_sample,), dtype=jnp.int32)
    d0 = jnp.full((P,), 1e30, dtype=jnp.float32)

    def body(i, st):
        idxs, dists = st
        last = idxs[i - 1]
        d = jnp.sum((pos - pos[last]) ** 2, axis=1)
        dists = jnp.minimum(dists, d)
        idxs = idxs.at[i].set(jnp.argmax(dists).astype(jnp.int32))
        return (idxs, dists)

    idxs, _ = jax.lax.fori_loop(1, n_sample, body, (idxs0, d0))
    return idxs


def _radius(pos, qpos, r, max_n=64):
    d2 = jnp.sum((qpos[:, None, :] - pos[None, :, :]) ** 2, axis=-1)
    within = d2 < r * r
    score = jnp.where(within, -d2, -1e30)
    vals, nbr = jax.lax.top_k(score, max_n)
    valid = vals > -1e29
    return nbr, valid


def _point_conv(layers, x, pos, qpos, nbr, valid):
    xj = x[nbr]
    rel = pos[nbr] - qpos[:, None, :]
    msg = _mlp(layers, jnp.concatenate([xj, rel], axis=-1))
    msg = jnp.where(valid[..., None], msg, -1e30)
    out = jnp.max(msg, axis=1)
    return jnp.where(jnp.any(valid, axis=1)[:, None], out, 0.0)


def _knn_interp(x, pos, pos_skip, k):
    d2 = jnp.sum((pos_skip[:, None, :] - pos[None, :, :]) ** 2, axis=-1)
    neg, idx = jax.lax.top_k(-d2, k)
    w = 1.0 / jnp.maximum(-neg, 1e-16)
    w = w / jnp.sum(w, axis=-1, keepdims=True)
    return jnp.sum(x[idx] * w[..., None], axis=1)


def _head_kernel(xin_ref, *refs):
    out_ref = refs[-1]
    ws = refs[:-1]
    h = xin_ref[...]
    i = 0
    for _ in range(3):
        W, b, g, be = ws[i], ws[i + 1], ws[i + 2], ws[i + 3]
        i += 4
        h = jnp.dot(h, W[...], preferred_element_type=jnp.float32) + b[...]
        h = jnp.maximum(h, 0.0)
        h = h * (_BN_S * g[...]) + be[...]
    W1, b1, W2, b2, W3, b3 = ws[i], ws[i + 1], ws[i + 2], ws[i + 3], ws[i + 4], ws[i + 5]
    h = jnp.maximum(jnp.dot(h, W1[...], preferred_element_type=jnp.float32) + b1[...], 0.0)
    h = jnp.dot(h, W2[...], preferred_element_type=jnp.float32) + b2[...]
    h = jnp.dot(h, W3[...], preferred_element_type=jnp.float32) + b3[...]
    out_ref[...] = h


def _head(xin, params):
    N = xin.shape[0]
    BLK = 2048
    ws = []
    for l in params["fp1"]:
        ws += [l["W"], l["b"].reshape(1, -1), l["gamma"].reshape(1, -1), l["beta"].reshape(1, -1)]
    for n in ("lin1", "lin2", "lin3"):
        ws += [params[n]["W"], params[n]["b"].reshape(1, -1)]
    grid = (N // BLK,)
    in_specs = [pl.BlockSpec((BLK, xin.shape[1]), lambda i: (i, 0))]
    for w in ws:
        in_specs.append(pl.BlockSpec(w.shape, lambda i: (0, 0)))
    out = pl.pallas_call(
        _head_kernel,
        grid=grid,
        in_specs=in_specs,
        out_specs=pl.BlockSpec((BLK, 13), lambda i: (i, 0)),
        out_shape=jax.ShapeDtypeStruct((N, 13), jnp.float32),
    )(xin, *ws)
    return out


def _mid_stage(params, x2, pos2):
    h = _mlp(params["sa3"], jnp.concatenate([x2, pos2], axis=1))
    x3 = jnp.max(h, axis=0, keepdims=True)
    xf = jnp.broadcast_to(x3, (_S2, x3.shape[1]))
    return _mlp(params["fp3"], jnp.concatenate([xf, x2], axis=1))


def kernel(x, pos, batch, params):
    B = 8
    P = x.shape[0] // B
    xb = x.reshape(B, P, -1)
    pb = pos.reshape(B, P, 3)
    pos1, pos2 = _fps_pallas(pb)
    x1 = _radius_conv(xb, pb, pos1, params["sa1"], _P, _S1, 0.2 * 0.2)
    x2 = _radius_conv(x1, pos1, pos2, params["sa2"], _S1, _S2, 0.4 * 0.4)
    xf3 = jax.vmap(lambda x2c, p2c: _mid_stage(params, x2c, p2c))(x2, pos2)
    xf2 = _knn_mlp(pos2, pos1, xf3, x1, params["fp2"], [], _S2, _S1)
    out = _knn_mlp(pos1, pb, xf2, xb, params["fp1"],
                   [params["lin1"], params["lin2"], params["lin3"]], _S1, _P)
    out = out.reshape(B * P, -1)
    zero = (batch[-1].astype(jnp.int32) + 1 - B).astype(out.dtype)
    return out + zero


# speedup vs baseline: 1.2099x; 1.2099x over previous
import functools

import jax
import jax.numpy as jnp
import numpy as np
from jax.experimental import pallas as pl
from jax.experimental.pallas import tpu as pltpu

_EPS_BN = 1e-5
_BN_S = 1.0 / np.sqrt(1.0 + _EPS_BN)
_B, _P, _S1, _S2 = 8, 2048, 410, 103


def _fps_kernel(px_ref, py_ref, pz_ref, px1_ref, py1_ref, pz1_ref,
                px2_ref, py2_ref, pz2_ref):
    px, py, pz = px_ref[...], py_ref[...], pz_ref[...]
    lane_p = jax.lax.broadcasted_iota(jnp.int32, (_B, _P), 1)
    lane_s1 = jax.lax.broadcasted_iota(jnp.int32, (_B, _S1), 1)

    def run_fps(ax_src, ay_src, az_src, n_pts, n_samp, lane_src, lane_dst):
        lx = ax_src[:, 0:1]
        ly = ay_src[:, 0:1]
        lz = az_src[:, 0:1]
        ox = jnp.where(lane_dst == 0, lx, 0.0)
        oy = jnp.where(lane_dst == 0, ly, 0.0)
        oz = jnp.where(lane_dst == 0, lz, 0.0)
        dists = jnp.full((_B, n_pts), 1e30, dtype=jnp.float32)

        def body(i, c):
            lx, ly, lz, dists, ox, oy, oz = c
            d = (ax_src - lx) ** 2 + (ay_src - ly) ** 2 + (az_src - lz) ** 2
            dists = jnp.minimum(dists, d)
            m = jnp.max(dists, axis=1, keepdims=True)
            sel = jnp.min(jnp.where(dists == m, lane_src, n_pts),
                          axis=1, keepdims=True)
            eq = lane_src == sel
            nlx = jnp.sum(jnp.where(eq, ax_src, 0.0), axis=1, keepdims=True)
            nly = jnp.sum(jnp.where(eq, ay_src, 0.0), axis=1, keepdims=True)
            nlz = jnp.sum(jnp.where(eq, az_src, 0.0), axis=1, keepdims=True)
            ox = jnp.where(lane_dst == i, nlx, ox)
            oy = jnp.where(lane_dst == i, nly, oy)
            oz = jnp.where(lane_dst == i, nlz, oz)
            return (nlx, nly, nlz, dists, ox, oy, oz)

        c = (lx, ly, lz, dists, ox, oy, oz)
        c = jax.lax.fori_loop(1, n_samp, body, c)
        return c[4], c[5], c[6]

    px1, py1, pz1 = run_fps(px, py, pz, _P, _S1, lane_p, lane_s1)
    px1_ref[...], py1_ref[...], pz1_ref[...] = px1, py1, pz1
    lane_s2 = jax.lax.broadcasted_iota(jnp.int32, (_B, _S2), 1)
    px2, py2, pz2 = run_fps(px1, py1, pz1, _S1, _S2, lane_s1, lane_s2)
    px2_ref[...], py2_ref[...], pz2_ref[...] = px2, py2, pz2


def _lane_cumsum(x, n):
    s = 1
    while s < n:
        shifted = jnp.concatenate(
            [jnp.zeros((x.shape[0], s), x.dtype), x[:, : n - s]], axis=1)
        x = x + shifted
        s *= 2
    return x


def _make_radius_conv_kernel(N, Q, QP, F, rr, dims, QB, K=64):
    n_layers = len(dims) - 1
    rr_bits = int(np.frombuffer(np.float32(rr).tobytes(), dtype=np.int32)[0])
    SENT = np.int32(2**31 - 1)

    def kern(*refs):
        px_r, py_r, pz_r, x_ref, qx_c, qy_c, qz_c = refs[:7]
        wrefs = refs[7:-1]
        out_ref = refs[-1]
        px = px_r[...].reshape(1, N)
        py = py_r[...].reshape(1, N)
        pz = pz_r[...].reshape(1, N)
        qx = qx_c[...].reshape(QP, 1)
        qy = qy_c[...].reshape(QP, 1)
        qz = qz_c[...].reshape(QP, 1)
        xs = x_ref[...].reshape(N, F)
        W1 = wrefs[0][...]
        b1 = wrefs[1][...]
        d2 = (qx - px) ** 2 + (qy - py) ** 2 + (qz - pz) ** 2
        key = jax.lax.bitcast_convert_type(d2, jnp.int32)
        within = key < rr_bits
        key = jnp.where(within, key, SENT)
        lo = jnp.zeros((QP, 1), jnp.int32)
        hi = jnp.full((QP, 1), rr_bits, jnp.int32)

        def bs_body(_, c):
            lo, hi = c
            mid = lo + (hi - lo) // 2
            cnt = jnp.sum((key <= mid).astype(jnp.int32), axis=1,
                          keepdims=True)
            ge = cnt >= K
            hi = jnp.where(ge, mid, hi)
            lo = jnp.where(ge, lo, mid + 1)
            return lo, hi

        lo, hi = jax.lax.fori_loop(0, 30, bs_body, (lo, hi))
        v64 = hi
        less = within & (key < v64)
        cnt_less = jnp.sum(less.astype(jnp.int32), axis=1, keepdims=True)
        tie = within & (key == v64)
        packed = (tie.astype(jnp.int32) << 16) | less.astype(jnp.int32)
        cs = _lane_cumsum(packed, N)
        excl_tie = (cs >> 16) - tie.astype(jnp.int32)
        excl_less = (cs & 0xFFFF) - less.astype(jnp.int32)
        need = K - cnt_less
        sel = less | (tie & (excl_tie < need))
        count = jnp.sum(sel.astype(jnp.int32), axis=1, keepdims=True)
        slot = excl_less + jnp.minimum(excl_tie, need)
        slot = jnp.where(sel, slot, -1)
        a = jnp.dot(xs, W1[:F, :], preferred_element_type=jnp.float32)
        a = a + px.reshape(N, 1) * W1[F, :].reshape(1, -1)
        a = a + py.reshape(N, 1) * W1[F + 1, :].reshape(1, -1)
        a = a + pz.reshape(N, 1) * W1[F + 2, :].reshape(1, -1)
        cq = b1 - (qx * W1[F, :].reshape(1, -1)
                   + qy * W1[F + 1, :].reshape(1, -1)
                   + qz * W1[F + 2, :].reshape(1, -1))
        s_iota = jax.lax.broadcasted_iota(jnp.int32, (1, K, 1), 1)
        gam = wrefs[2][...]
        bet = wrefs[3][...]
        for qb in range(QP // QB):
            sl = slot[qb * QB:(qb + 1) * QB, :]
            m3 = (sl[:, None, :] == s_iota).astype(jnp.float32)
            g = jax.lax.dot_general(
                m3, a, (((2,), (0,)), ((), ())),
                preferred_element_type=jnp.float32)
            h = g + cq[qb * QB:(qb + 1) * QB, None, :]
            h = jnp.maximum(h, 0.0) * (_BN_S * gam) + bet
            for li in range(1, n_layers):
                W = wrefs[4 * li][...]
                b = wrefs[4 * li + 1][...]
                gm = wrefs[4 * li + 2][...]
                bt = wrefs[4 * li + 3][...]
                h = jax.lax.dot_general(
                    h, W, (((2,), (0,)), ((), ())),
                    preferred_element_type=jnp.float32) + b
                h = jnp.maximum(h, 0.0) * (_BN_S * gm) + bt
            cb = count[qb * QB:(qb + 1) * QB, :]
            vmask = s_iota < cb[:, None, :]
            h = jnp.where(vmask, h, -1e30)
            h = jnp.max(h, axis=1)
            h = jnp.where(cb > 0, h, 0.0)
            out_ref[0, qb * QB:(qb + 1) * QB, :] = h

    return kern


def _radius_conv(pb_feat, pb_pos, q_pos, layers, N, Q, rr, QB=16):
    B = pb_feat.shape[0]
    F = pb_feat.shape[2]
    dims = [F + 3] + [l["W"].shape[1] for l in layers]
    QP = -(-Q // QB) * QB
    pad_q = QP - Q
    qp = jnp.pad(q_pos, ((0, 0), (0, pad_q), (0, 0)), constant_values=1e6)
    ins = [pb_pos[:, None, :, 0], pb_pos[:, None, :, 1], pb_pos[:, None, :, 2],
           pb_feat,
           qp[..., 0:1], qp[..., 1:2], qp[..., 2:3]]
    wlist = []
    for l in layers:
        wlist += [l["W"], l["b"].reshape(1, -1), l["gamma"].reshape(1, -1),
                  l["beta"].reshape(1, -1)]
    in_specs = [
        pl.BlockSpec((1, 1, N), lambda i: (i, 0, 0)),
        pl.BlockSpec((1, 1, N), lambda i: (i, 0, 0)),
        pl.BlockSpec((1, 1, N), lambda i: (i, 0, 0)),
        pl.BlockSpec((1, N, F), lambda i: (i, 0, 0)),
        pl.BlockSpec((1, QP, 1), lambda i: (i, 0, 0)),
        pl.BlockSpec((1, QP, 1), lambda i: (i, 0, 0)),
        pl.BlockSpec((1, QP, 1), lambda i: (i, 0, 0)),
    ] + [pl.BlockSpec(w.shape, lambda i: (0, 0)) for w in wlist]

    kern = _make_radius_conv_kernel(N, Q, QP, F, rr, dims, QB)

    out = pl.pallas_call(
        kern,
        grid=(B,),
        in_specs=in_specs,
        out_specs=pl.BlockSpec((1, QP, dims[-1]), lambda i: (i, 0, 0)),
        out_shape=jax.ShapeDtypeStruct((B, QP, dims[-1]), jnp.float32),
    )(*ins, *wlist)
    return out[:, :Q, :]


def _make_knn_mlp_kernel(N, Q, QP, D, S, n_bn, n_lin):
    def kern(*refs):
        sx_r, sy_r, sz_r, f_ref, qx_c, qy_c, qz_c, skip_ref = refs[:8]
        wrefs = refs[8:-1]
        out_ref = refs[-1]
        sx = sx_r[...].reshape(1, N)
        sy = sy_r[...].reshape(1, N)
        sz = sz_r[...].reshape(1, N)
        qx = qx_c[...].reshape(QP, 1)
        qy = qy_c[...].reshape(QP, 1)
        qz = qz_c[...].reshape(QP, 1)
        feat = f_ref[...].reshape(N, D)
        skip = skip_ref[...].reshape(QP, S)
        lane = jax.lax.broadcasted_iota(jnp.int32, (QP, N), 1)
        d2 = (qx - sx) ** 2 + (qy - sy) ** 2 + (qz - sz) ** 2
        xs, ws = [], []
        for _ in range(3):
            m = jnp.min(d2, axis=1, keepdims=True)
            idx = jnp.min(jnp.where(d2 == m, lane, N), axis=1, keepdims=True)
            eq = (lane == idx).astype(jnp.float32)
            xs.append(jnp.dot(eq, feat, preferred_element_type=jnp.float32))
            ws.append(1.0 / jnp.maximum(m, 1e-16))
            d2 = jnp.where(lane == idx, 3e38, d2)
        wsum = (ws[0] + ws[1]) + ws[2]
        h = (xs[0] * (ws[0] / wsum) + xs[1] * (ws[1] / wsum)) \
            + xs[2] * (ws[2] / wsum)
        i = 0
        for li in range(n_bn):
            W, b, g, be = (wrefs[i][...], wrefs[i + 1][...],
                           wrefs[i + 2][...], wrefs[i + 3][...])
            i += 4
            if li == 0:
                z = jnp.dot(h, W[:D, :], preferred_element_type=jnp.float32) \
                    + jnp.dot(skip, W[D:, :],
                              preferred_element_type=jnp.float32) + b
            else:
                z = jnp.dot(h, W, preferred_element_type=jnp.float32) + b
            h = jnp.maximum(z, 0.0) * (_BN_S * g) + be
        for li in range(n_lin):
            W, b = wrefs[i][...], wrefs[i + 1][...]
            i += 2
            h = jnp.dot(h, W, preferred_element_type=jnp.float32) + b
            if li == 0:
                h = jnp.maximum(h, 0.0)
        out_ref[...] = h.reshape(1, QP, h.shape[1])

    return kern


def _knn_mlp(src_pos, dst_pos, feat, skip, bn_layers, lin_layers, N, Q):
    B, D = feat.shape[0], feat.shape[2]
    S = skip.shape[2]
    QP = -(-Q // 8) * 8
    pad_q = QP - Q
    qp = jnp.pad(dst_pos, ((0, 0), (0, pad_q), (0, 0)), constant_values=1e6)
    skp = jnp.pad(skip, ((0, 0), (0, pad_q), (0, 0)))
    ins = [src_pos[:, None, :, 0], src_pos[:, None, :, 1],
           src_pos[:, None, :, 2], feat,
           qp[..., 0:1], qp[..., 1:2], qp[..., 2:3], skp]
    wlist = []
    for l in bn_layers:
        wlist += [l["W"], l["b"].reshape(1, -1), l["gamma"].reshape(1, -1),
                  l["beta"].reshape(1, -1)]
    for l in lin_layers:
        wlist += [l["W"], l["b"].reshape(1, -1)]
    dout = (lin_layers[-1]["W"].shape[1] if lin_layers
            else bn_layers[-1]["W"].shape[1])
    in_specs = [
        pl.BlockSpec((1, 1, N), lambda i: (i, 0, 0)),
        pl.BlockSpec((1, 1, N), lambda i: (i, 0, 0)),
        pl.BlockSpec((1, 1, N), lambda i: (i, 0, 0)),
        pl.BlockSpec((1, N, D), lambda i: (i, 0, 0)),
        pl.BlockSpec((1, QP, 1), lambda i: (i, 0, 0)),
        pl.BlockSpec((1, QP, 1), lambda i: (i, 0, 0)),
        pl.BlockSpec((1, QP, 1), lambda i: (i, 0, 0)),
        pl.BlockSpec((1, QP, S), lambda i: (i, 0, 0)),
    ] + [pl.BlockSpec(w.shape, lambda i: (0, 0)) for w in wlist]
    kern = _make_knn_mlp_kernel(N, Q, QP, D, S, len(bn_layers),
                                len(lin_layers))
    out = pl.pallas_call(
        kern,
        grid=(B,),
        in_specs=in_specs,
        out_specs=pl.BlockSpec((1, QP, dout), lambda i: (i, 0, 0)),
        out_shape=jax.ShapeDtypeStruct((B, QP, dout), jnp.float32),
    )(*ins, *wlist)
    return out[:, :Q, :]


def _fps_pallas(pb):
    px = pb[..., 0]
    py = pb[..., 1]
    pz = pb[..., 2]
    outs = pl.pallas_call(
        _fps_kernel,
        out_shape=[jax.ShapeDtypeStruct((_B, _S1), jnp.float32)] * 3
        + [jax.ShapeDtypeStruct((_B, _S2), jnp.float32)] * 3,
    )(px, py, pz)
    pos1 = jnp.stack(outs[0:3], axis=-1)
    pos2 = jnp.stack(outs[3:6], axis=-1)
    return pos1, pos2


def _mlp(layers, x):
    for l in layers:
        x = x @ l["W"] + l["b"]
        x = jax.nn.relu(x)
        x = x * (_BN_S * l["gamma"]) + l["beta"]
    return x


def _fps(pos, n_sample):
    P = pos.shape[0]
    idxs0 = jnp.zeros((n_sample,), dtype=jnp.int32)
    d0 = jnp.full((P,), 1e30, dtype=jnp.float32)

    def body(i, st):
        idxs, dists = st
        last = idxs[i - 1]
        d = jnp.sum((pos - pos[last]) ** 2, axis=1)
        dists = jnp.minimum(dists, d)
        idxs = idxs.at[i].set(jnp.argmax(dists).astype(jnp.int32))
        return (idxs, dists)

    idxs, _ = jax.lax.fori_loop(1, n_sample, body, (idxs0, d0))
    return idxs


def _radius(pos, qpos, r, max_n=64):
    d2 = jnp.sum((qpos[:, None, :] - pos[None, :, :]) ** 2, axis=-1)
    within = d2 < r * r
    score = jnp.where(within, -d2, -1e30)
    vals, nbr = jax.lax.top_k(score, max_n)
    valid = vals > -1e29
    return nbr, valid


def _point_conv(layers, x, pos, qpos, nbr, valid):
    xj = x[nbr]
    rel = pos[nbr] - qpos[:, None, :]
    msg = _mlp(layers, jnp.concatenate([xj, rel], axis=-1))
    msg = jnp.where(valid[..., None], msg, -1e30)
    out = jnp.max(msg, axis=1)
    return jnp.where(jnp.any(valid, axis=1)[:, None], out, 0.0)


def _knn_interp(x, pos, pos_skip, k):
    d2 = jnp.sum((pos_skip[:, None, :] - pos[None, :, :]) ** 2, axis=-1)
    neg, idx = jax.lax.top_k(-d2, k)
    w = 1.0 / jnp.maximum(-neg, 1e-16)
    w = w / jnp.sum(w, axis=-1, keepdims=True)
    return jnp.sum(x[idx] * w[..., None], axis=1)


def _head_kernel(xin_ref, *refs):
    out_ref = refs[-1]
    ws = refs[:-1]
    h = xin_ref[...]
    i = 0
    for _ in range(3):
        W, b, g, be = ws[i], ws[i + 1], ws[i + 2], ws[i + 3]
        i += 4
        h = jnp.dot(h, W[...], preferred_element_type=jnp.float32) + b[...]
        h = jnp.maximum(h, 0.0)
        h = h * (_BN_S * g[...]) + be[...]
    W1, b1, W2, b2, W3, b3 = ws[i], ws[i + 1], ws[i + 2], ws[i + 3], ws[i + 4], ws[i + 5]
    h = jnp.maximum(jnp.dot(h, W1[...], preferred_element_type=jnp.float32) + b1[...], 0.0)
    h = jnp.dot(h, W2[...], preferred_element_type=jnp.float32) + b2[...]
    h = jnp.dot(h, W3[...], preferred_element_type=jnp.float32) + b3[...]
    out_ref[...] = h


def _head(xin, params):
    N = xin.shape[0]
    BLK = 2048
    ws = []
    for l in params["fp1"]:
        ws += [l["W"], l["b"].reshape(1, -1), l["gamma"].reshape(1, -1), l["beta"].reshape(1, -1)]
    for n in ("lin1", "lin2", "lin3"):
        ws += [params[n]["W"], params[n]["b"].reshape(1, -1)]
    grid = (N // BLK,)
    in_specs = [pl.BlockSpec((BLK, xin.shape[1]), lambda i: (i, 0))]
    for w in ws:
        in_specs.append(pl.BlockSpec(w.shape, lambda i: (0, 0)))
    out = pl.pallas_call(
        _head_kernel,
        grid=grid,
        in_specs=in_specs,
        out_specs=pl.BlockSpec((BLK, 13), lambda i: (i, 0)),
        out_shape=jax.ShapeDtypeStruct((N, 13), jnp.float32),
    )(xin, *ws)
    return out


def _mid_stage(params, x2, pos2):
    h = _mlp(params["sa3"], jnp.concatenate([x2, pos2], axis=1))
    x3 = jnp.max(h, axis=0, keepdims=True)
    xf = jnp.broadcast_to(x3, (_S2, x3.shape[1]))
    return _mlp(params["fp3"], jnp.concatenate([xf, x2], axis=1))


def kernel(x, pos, batch, params):
    B = 8
    P = x.shape[0] // B
    xb = x.reshape(B, P, -1)
    pb = pos.reshape(B, P, 3)
    pos1, pos2 = _fps_pallas(pb)
    x1 = _radius_conv(xb, pb, pos1, params["sa1"], _P, _S1, 0.2 * 0.2)
    x2 = _radius_conv(x1, pos1, pos2, params["sa2"], _S1, _S2, 0.4 * 0.4)
    xf3 = jax.vmap(lambda x2c, p2c: _mid_stage(params, x2c, p2c))(x2, pos2)
    xf2 = _knn_mlp(pos2, pos1, xf3, x1, params["fp2"], [], _S2, _S1)
    out = _knn_mlp(pos1, pb, xf2, xb, params["fp1"],
                   [params["lin1"], params["lin2"], params["lin3"]], _S1, _P)
    out = out.reshape(B * P, -1)
    zero = (batch[-1].astype(jnp.int32) + 1 - B).astype(out.dtype)
    return out + zero
